# baseline (device time: 12059 ns/iter reference)
import jax
import jax.numpy as jnp
from jax import lax
from jax.experimental import pallas as pl
from jax.experimental.pallas import tpu as pltpu

K = 16


def kernel(x):
    m, n = x.shape
    half = m // 2
    c = half // K

    def body(x_hbm, out_hbm, xstage, sbuf, r1buf, pbuf, r2buf,
             send1, recv1, send2, recv2, in_sem, osem1, osem2):
        my_x = lax.axis_index("x")
        my_y = lax.axis_index("y")
        xn = (1 - my_x, my_y)
        yn = (my_x, 1 - my_y)

        base = my_y * half
        other = (1 - my_y) * half

        incp = pltpu.make_async_copy(
            x_hbm.at[pl.ds(base, half), :], xstage, in_sem
        )
        incp.start()

        barrier_sem = pltpu.get_barrier_semaphore()
        for nbr in (xn, yn):
            pl.semaphore_signal(
                barrier_sem, inc=1, device_id=nbr,
                device_id_type=pl.DeviceIdType.MESH,
            )
        pl.semaphore_wait(barrier_sem, 2)
        incp.wait()

        rdma1 = []
        for k in range(K):
            ds = pl.ds(k * c, c)
            sbuf[ds, :] = xstage[ds, :].astype(jnp.bfloat16)
            r = pltpu.make_async_remote_copy(
                src_ref=sbuf.at[ds, :],
                dst_ref=r1buf.at[ds, :],
                send_sem=send1.at[k],
                recv_sem=recv1.at[k],
                device_id=xn,
                device_id_type=pl.DeviceIdType.MESH,
            )
            r.start()
            rdma1.append(r)

        rdma2 = []
        ocp1 = []
        for k in range(K):
            ds = pl.ds(k * c, c)
            rdma1[k].wait_recv()
            pbuf[ds, :] = sbuf[ds, :] + r1buf[ds, :]
            r = pltpu.make_async_remote_copy(
                src_ref=pbuf.at[ds, :],
                dst_ref=r2buf.at[ds, :],
                send_sem=send2.at[k],
                recv_sem=recv2.at[k],
                device_id=yn,
                device_id_type=pl.DeviceIdType.MESH,
            )
            r.start()
            rdma2.append(r)
            o = pltpu.make_async_copy(
                pbuf.at[ds, :],
                out_hbm.at[pl.ds(base + k * c, c), :],
                osem1.at[k],
            )
            o.start()
            ocp1.append(o)

        ocp2 = []
        for k in range(K):
            ds = pl.ds(k * c, c)
            rdma2[k].wait_recv()
            o = pltpu.make_async_copy(
                r2buf.at[ds, :],
                out_hbm.at[pl.ds(other + k * c, c), :],
                osem2.at[k],
            )
            o.start()
            ocp2.append(o)

        for k in range(K):
            ocp1[k].wait()
            ocp2[k].wait()
            rdma1[k].wait_send()
            rdma2[k].wait_send()

    return pl.pallas_call(
        body,
        out_shape=jax.ShapeDtypeStruct((m, n), jnp.bfloat16),
        in_specs=[pl.BlockSpec(memory_space=pl.ANY)],
        out_specs=pl.BlockSpec(memory_space=pl.ANY),
        scratch_shapes=[
            pltpu.VMEM((half, n), jnp.float32),
            pltpu.VMEM((half, n), jnp.bfloat16),
            pltpu.VMEM((half, n), jnp.bfloat16),
            pltpu.VMEM((half, n), jnp.bfloat16),
            pltpu.VMEM((half, n), jnp.bfloat16),
            pltpu.SemaphoreType.DMA((K,)),
            pltpu.SemaphoreType.DMA((K,)),
            pltpu.SemaphoreType.DMA((K,)),
            pltpu.SemaphoreType.DMA((K,)),
            pltpu.SemaphoreType.DMA,
            pltpu.SemaphoreType.DMA((K,)),
            pltpu.SemaphoreType.DMA((K,)),
        ],
        compiler_params=pltpu.CompilerParams(collective_id=0),
    )(x)


# device time: 11772 ns/iter; 1.0244x vs baseline; 1.0244x over previous
import jax
import jax.numpy as jnp
from jax import lax
from jax.experimental import pallas as pl
from jax.experimental.pallas import tpu as pltpu

K = 8


def kernel(x):
    m, n = x.shape
    half = m // 2
    c = half // K

    def body(x_hbm, out_hbm, xstage, sbuf, r1buf, pbuf, r2buf,
             send1, recv1, send2, recv2, in_sem, osem1, osem2):
        my_x = lax.axis_index("x")
        my_y = lax.axis_index("y")
        xn = (1 - my_x, my_y)
        yn = (my_x, 1 - my_y)

        base = my_y * half
        other = (1 - my_y) * half

        incp = pltpu.make_async_copy(
            x_hbm.at[pl.ds(base, half), :], xstage, in_sem
        )
        incp.start()

        barrier_sem = pltpu.get_barrier_semaphore()
        for nbr in (xn, yn):
            pl.semaphore_signal(
                barrier_sem, inc=1, device_id=nbr,
                device_id_type=pl.DeviceIdType.MESH,
            )
        pl.semaphore_wait(barrier_sem, 2)
        incp.wait()

        rdma1 = []
        for k in range(K):
            ds = pl.ds(k * c, c)
            sbuf[ds, :] = xstage[ds, :].astype(jnp.bfloat16)
            r = pltpu.make_async_remote_copy(
                src_ref=sbuf.at[ds, :],
                dst_ref=r1buf.at[ds, :],
                send_sem=send1.at[k],
                recv_sem=recv1.at[k],
                device_id=xn,
                device_id_type=pl.DeviceIdType.MESH,
            )
            r.start()
            rdma1.append(r)

        rdma2 = []
        ocp1 = []
        for k in range(K):
            ds = pl.ds(k * c, c)
            rdma1[k].wait_recv()
            pbuf[ds, :] = sbuf[ds, :] + r1buf[ds, :]
            r = pltpu.make_async_remote_copy(
                src_ref=pbuf.at[ds, :],
                dst_ref=r2buf.at[ds, :],
                send_sem=send2.at[k],
                recv_sem=recv2.at[k],
                device_id=yn,
                device_id_type=pl.DeviceIdType.MESH,
            )
            r.start()
            rdma2.append(r)
            o = pltpu.make_async_copy(
                pbuf.at[ds, :],
                out_hbm.at[pl.ds(base + k * c, c), :],
                osem1.at[k],
            )
            o.start()
            ocp1.append(o)

        ocp2 = []
        for k in range(K):
            ds = pl.ds(k * c, c)
            rdma2[k].wait_recv()
            o = pltpu.make_async_copy(
                r2buf.at[ds, :],
                out_hbm.at[pl.ds(other + k * c, c), :],
                osem2.at[k],
            )
            o.start()
            ocp2.append(o)

        for k in range(K):
            ocp1[k].wait()
            ocp2[k].wait()
            rdma1[k].wait_send()
            rdma2[k].wait_send()

    return pl.pallas_call(
        body,
        out_shape=jax.ShapeDtypeStruct((m, n), jnp.bfloat16),
        in_specs=[pl.BlockSpec(memory_space=pl.ANY)],
        out_specs=pl.BlockSpec(memory_space=pl.ANY),
        scratch_shapes=[
            pltpu.VMEM((half, n), jnp.float32),
            pltpu.VMEM((half, n), jnp.bfloat16),
            pltpu.VMEM((half, n), jnp.bfloat16),
            pltpu.VMEM((half, n), jnp.bfloat16),
            pltpu.VMEM((half, n), jnp.bfloat16),
            pltpu.SemaphoreType.DMA((K,)),
            pltpu.SemaphoreType.DMA((K,)),
            pltpu.SemaphoreType.DMA((K,)),
            pltpu.SemaphoreType.DMA((K,)),
            pltpu.SemaphoreType.DMA,
            pltpu.SemaphoreType.DMA((K,)),
            pltpu.SemaphoreType.DMA((K,)),
        ],
        compiler_params=pltpu.CompilerParams(collective_id=0),
    )(x)


# device time: 11727 ns/iter; 1.0283x vs baseline; 1.0038x over previous
import jax
import jax.numpy as jnp
from jax import lax
from jax.experimental import pallas as pl
from jax.experimental.pallas import tpu as pltpu

K = 8


def kernel(x):
    m, n = x.shape
    half = m // 2
    c = half // K

    def body(x_hbm, out_hbm, xstage, sbuf, r1buf, pbuf, r2buf,
             send1, recv1, send2, recv2, in_sem, osem1, osem2, ybar):
        my_x = lax.axis_index("x")
        my_y = lax.axis_index("y")
        xn = (1 - my_x, my_y)
        yn = (my_x, 1 - my_y)

        base = my_y * half
        other = (1 - my_y) * half

        incp = pltpu.make_async_copy(
            x_hbm.at[pl.ds(base, half), :], xstage, in_sem
        )
        incp.start()

        barrier_sem = pltpu.get_barrier_semaphore()
        pl.semaphore_signal(
            barrier_sem, inc=1, device_id=xn,
            device_id_type=pl.DeviceIdType.MESH,
        )
        pl.semaphore_signal(
            ybar, inc=1, device_id=yn,
            device_id_type=pl.DeviceIdType.MESH,
        )
        pl.semaphore_wait(barrier_sem, 1)
        incp.wait()

        rdma1 = []
        for k in range(K):
            ds = pl.ds(k * c, c)
            sbuf[ds, :] = xstage[ds, :].astype(jnp.bfloat16)
            r = pltpu.make_async_remote_copy(
                src_ref=sbuf.at[ds, :],
                dst_ref=r1buf.at[ds, :],
                send_sem=send1.at[k],
                recv_sem=recv1.at[k],
                device_id=xn,
                device_id_type=pl.DeviceIdType.MESH,
            )
            r.start()
            rdma1.append(r)

        rdma2 = []
        ocp1 = []
        for k in range(K):
            ds = pl.ds(k * c, c)
            rdma1[k].wait_recv()
            pbuf[ds, :] = sbuf[ds, :] + r1buf[ds, :]
            if k == 0:
                pl.semaphore_wait(ybar, 1)
            r = pltpu.make_async_remote_copy(
                src_ref=pbuf.at[ds, :],
                dst_ref=r2buf.at[ds, :],
                send_sem=send2.at[k],
                recv_sem=recv2.at[k],
                device_id=yn,
                device_id_type=pl.DeviceIdType.MESH,
            )
            r.start()
            rdma2.append(r)
            o = pltpu.make_async_copy(
                pbuf.at[ds, :],
                out_hbm.at[pl.ds(base + k * c, c), :],
                osem1.at[k],
            )
            o.start()
            ocp1.append(o)

        ocp2 = []
        for k in range(K):
            ds = pl.ds(k * c, c)
            rdma2[k].wait_recv()
            o = pltpu.make_async_copy(
                r2buf.at[ds, :],
                out_hbm.at[pl.ds(other + k * c, c), :],
                osem2.at[k],
            )
            o.start()
            ocp2.append(o)

        for k in range(K):
            ocp1[k].wait()
            ocp2[k].wait()
            rdma1[k].wait_send()
            rdma2[k].wait_send()

    return pl.pallas_call(
        body,
        out_shape=jax.ShapeDtypeStruct((m, n), jnp.bfloat16),
        in_specs=[pl.BlockSpec(memory_space=pl.ANY)],
        out_specs=pl.BlockSpec(memory_space=pl.ANY),
        scratch_shapes=[
            pltpu.VMEM((half, n), jnp.float32),
            pltpu.VMEM((half, n), jnp.bfloat16),
            pltpu.VMEM((half, n), jnp.bfloat16),
            pltpu.VMEM((half, n), jnp.bfloat16),
            pltpu.VMEM((half, n), jnp.bfloat16),
            pltpu.SemaphoreType.DMA((K,)),
            pltpu.SemaphoreType.DMA((K,)),
            pltpu.SemaphoreType.DMA((K,)),
            pltpu.SemaphoreType.DMA((K,)),
            pltpu.SemaphoreType.DMA,
            pltpu.SemaphoreType.DMA((K,)),
            pltpu.SemaphoreType.DMA((K,)),
            pltpu.SemaphoreType.REGULAR,
        ],
        compiler_params=pltpu.CompilerParams(collective_id=0),
    )(x)
